# baseline (speedup 1.0000x reference)
import functools

import jax
import jax.numpy as jnp
from jax import lax
from jax.experimental import pallas as pl
from jax.experimental.pallas import tpu as pltpu
from jax.experimental.pallas import tpu_sc as plsc

_N = 10000
_D = 128
_E = 320000
_B = 2 * _E
_NW = 32
_B_PER_W = _B // _NW
_CHUNK = 200
_NSTEPS = _B_PER_W // _CHUNK
_NBUF = 4
_PD = 2
assert _B_PER_W % _CHUNK == 0 and _CHUNK % 8 == 0
assert _NSTEPS % _NBUF == 0

_mesh = plsc.VectorSubcoreMesh(core_axis_name="c", subcore_axis_name="s")


@functools.partial(
    pl.kernel,
    out_type=jax.ShapeDtypeStruct((_B, _D), jnp.float32),
    mesh=_mesh,
    scratch_types=[
        pltpu.VMEM((_B_PER_W,), jnp.int32),
        [pltpu.VMEM((_CHUNK, _D), jnp.float32)] * _NBUF,
        [pltpu.SemaphoreType.DMA] * _NBUF,
        [pltpu.SemaphoreType.DMA] * _NBUF,
    ],
)
def _gather_rows(table_hbm, idx_hbm, out_hbm, idx_v, rows_v, gsems, ssems):
    wid = lax.axis_index("s") * 2 + lax.axis_index("c")
    base = wid * _B_PER_W

    pltpu.sync_copy(idx_hbm.at[pl.ds(base, _B_PER_W)], idx_v)

    def start_gather(step, b):
        pltpu.async_copy(
            table_hbm.at[idx_v.at[pl.ds(step * _CHUNK, _CHUNK)]],
            rows_v[b], gsems[b])

    def wait_gather(step, b):
        pltpu.make_async_copy(
            table_hbm.at[idx_v.at[pl.ds(step * _CHUNK, _CHUNK)]],
            rows_v[b], gsems[b]).wait()

    def start_store(step, b):
        off = base + step * _CHUNK
        pltpu.async_copy(rows_v[b], out_hbm.at[pl.ds(off, _CHUNK)], ssems[b])

    def wait_store(step, b):
        off = base + step * _CHUNK
        pltpu.make_async_copy(rows_v[b], out_hbm.at[pl.ds(off, _CHUNK)],
                              ssems[b]).wait()

    start_gather(0, 0)
    start_gather(1, 1)

    @pl.loop(0, _NSTEPS, step=_NBUF)
    def _steps(i):
        for b in range(_NBUF):
            step = i + b
            wait_gather(step, b)
            start_store(step, b)
            bp = (b + _PD) % _NBUF

            @pl.when(step + _PD < _NSTEPS)
            def _prefetch():
                @pl.when(step >= _PD)
                def _reuse_wait():
                    wait_store(step - _PD, bp)

                start_gather(step + _PD, bp)

    for b2 in range(_NBUF):
        step = _NSTEPS - _NBUF + b2
        wait_store(step, step % _NBUF)


@jax.jit
def kernel(vertices, edges, edge_features, edge_matrices):
    del edge_features, edge_matrices
    idx = edges.reshape(_B)
    out = _gather_rows(vertices, idx)
    return out.reshape(2, _E, _D)


# speedup vs baseline: 1.5405x; 1.5405x over previous
import functools

import jax
import jax.numpy as jnp
from jax import lax
from jax.experimental import pallas as pl
from jax.experimental.pallas import tpu as pltpu
from jax.experimental.pallas import tpu_sc as plsc

_N = 10000
_D = 128
_E = 320000
_B = 2 * _E
_NW = 32
_B_PER_W = _B // _NW
_CHUNK = 40
_NSTEPS = _B_PER_W // _CHUNK
_NBUF = 4
_PD = 2
assert _B_PER_W % _CHUNK == 0 and _CHUNK % 8 == 0
assert _NSTEPS % _NBUF == 0

_mesh = plsc.VectorSubcoreMesh(core_axis_name="c", subcore_axis_name="s")


@functools.partial(
    pl.kernel,
    out_type=jax.ShapeDtypeStruct((_B, _D), jnp.float32),
    mesh=_mesh,
    scratch_types=[
        pltpu.VMEM_SHARED((_N, _D), jnp.float32),
        pltpu.VMEM((_B_PER_W,), jnp.int32),
        [pltpu.VMEM((_CHUNK, _D), jnp.float32)] * _NBUF,
        [pltpu.SemaphoreType.DMA] * _NBUF,
        [pltpu.SemaphoreType.DMA] * _NBUF,
    ],
)
def _gather_rows(table_hbm, idx_hbm, out_hbm, table_sp, idx_v, rows_v,
                 gsems, ssems):
    s = lax.axis_index("s")
    wid = s * 2 + lax.axis_index("c")
    base = wid * _B_PER_W

    pltpu.sync_copy(idx_hbm.at[pl.ds(base, _B_PER_W)], idx_v)

    rows_main = (_N // 16) // 8 * 8
    rem_start = rows_main * 16
    sbase = s * rows_main
    n_full = rows_main // _CHUNK
    stage_chunks = [(k * _CHUNK, _CHUNK) for k in range(n_full)]
    stage_chunks.append((n_full * _CHUNK, rows_main - n_full * _CHUNK))
    for off, sz in stage_chunks:
        pltpu.sync_copy(table_hbm.at[pl.ds(sbase + off, sz)],
                        rows_v[0].at[pl.ds(0, sz)])
        pltpu.sync_copy(rows_v[0].at[pl.ds(0, sz)],
                        table_sp.at[pl.ds(sbase + off, sz)])

    @pl.when(s == 0)
    def _copy_tail():
        pltpu.sync_copy(table_hbm.at[pl.ds(rem_start, _N - rem_start)],
                        rows_v[1].at[pl.ds(0, _N - rem_start)])
        pltpu.sync_copy(rows_v[1].at[pl.ds(0, _N - rem_start)],
                        table_sp.at[pl.ds(rem_start, _N - rem_start)])

    plsc.subcore_barrier()

    def start_gather(step, b):
        pltpu.async_copy(
            table_sp.at[idx_v.at[pl.ds(step * _CHUNK, _CHUNK)]],
            rows_v[b], gsems[b])

    def wait_gather(step, b):
        pltpu.make_async_copy(
            table_sp.at[idx_v.at[pl.ds(step * _CHUNK, _CHUNK)]],
            rows_v[b], gsems[b]).wait()

    def start_store(step, b):
        off = base + step * _CHUNK
        pltpu.async_copy(rows_v[b], out_hbm.at[pl.ds(off, _CHUNK)], ssems[b])

    def wait_store(step, b):
        off = base + step * _CHUNK
        pltpu.make_async_copy(rows_v[b], out_hbm.at[pl.ds(off, _CHUNK)],
                              ssems[b]).wait()

    start_gather(0, 0)
    start_gather(1, 1)

    @pl.loop(0, _NSTEPS, step=_NBUF)
    def _steps(i):
        for b in range(_NBUF):
            step = i + b
            wait_gather(step, b)
            start_store(step, b)
            bp = (b + _PD) % _NBUF

            @pl.when(step + _PD < _NSTEPS)
            def _prefetch():
                @pl.when(step >= _PD)
                def _reuse_wait():
                    wait_store(step - _PD, bp)

                start_gather(step + _PD, bp)

    for b2 in range(_NBUF):
        step = _NSTEPS - _NBUF + b2
        wait_store(step, step % _NBUF)


@jax.jit
def kernel(vertices, edges, edge_features, edge_matrices):
    del edge_features, edge_matrices
    idx = edges.reshape(_B)
    out = _gather_rows(vertices, idx)
    return out.reshape(2, _E, _D)
